# Initial kernel scaffold; baseline (speedup 1.0000x reference)
import jax
import jax.numpy as jnp
from jax.experimental import pallas as pl


def kernel(seq, coords, seq_id):
    raise NotImplementedError("write your pallas kernel here")


# speedup vs baseline: 1.9835x; 1.9835x over previous
import functools
import math

import jax
import jax.numpy as jnp
import numpy as np
from jax import lax
from jax.experimental import pallas as pl
from jax.experimental.pallas import tpu as pltpu
from jax.experimental.pallas import tpu_sc as plsc

D_MODEL = 1024
MAX_LEN = 5000
PE_DIM = D_MODEL // 4


def _pe_table() -> np.ndarray:
    position = np.arange(MAX_LEN, dtype=np.float32)[:, None]
    div_term = np.exp(
        np.arange(0, PE_DIM, 2).astype(np.float32) * (-math.log(10000.0) / PE_DIM)
    )
    pe = np.zeros((MAX_LEN, PE_DIM), dtype=np.float32)
    pe[:, 0::2] = np.sin(position * div_term)
    pe[:, 1::2] = np.cos(position * div_term)
    return pe


_PE = _pe_table()

_CHUNK = 64


def _sc_gather(pe, idx, n_rows):
    info = plsc.get_sparse_core_info()
    nc, ns = info.num_cores, info.num_subcores
    nw = nc * ns
    rows_per_w = n_rows // nw
    assert rows_per_w * nw == n_rows
    n_full = rows_per_w // _CHUNK
    tail = rows_per_w - n_full * _CHUNK
    assert tail % 8 == 0 and rows_per_w % 8 == 0

    mesh = plsc.VectorSubcoreMesh(core_axis_name="c", subcore_axis_name="s")

    scratch = [
        pltpu.VMEM((_CHUNK,), jnp.int32),
        pltpu.VMEM((_CHUNK, PE_DIM), jnp.float32),
        pltpu.SemaphoreType.DMA,
    ]
    if tail:
        scratch += [
            pltpu.VMEM((tail,), jnp.int32),
            pltpu.VMEM((tail, PE_DIM), jnp.float32),
        ]

    @functools.partial(
        pl.kernel,
        mesh=mesh,
        out_type=jax.ShapeDtypeStruct((n_rows, PE_DIM), jnp.float32),
        scratch_types=scratch,
    )
    def k(pe_hbm, idx_hbm, out_hbm, idx_v, rows_v, sem, *tail_refs):
        wid = lax.axis_index("s") * nc + lax.axis_index("c")
        w_base = wid * rows_per_w

        def body(t, carry):
            base = w_base + t * _CHUNK
            pltpu.sync_copy(idx_hbm.at[pl.ds(base, _CHUNK)], idx_v)
            pltpu.async_copy(pe_hbm.at[idx_v], rows_v, sem).wait()
            pltpu.sync_copy(rows_v, out_hbm.at[pl.ds(base, _CHUNK)])
            return carry

        lax.fori_loop(0, n_full, body, 0)

        if tail:
            idx_t, rows_t = tail_refs
            base = w_base + n_full * _CHUNK
            pltpu.sync_copy(idx_hbm.at[pl.ds(base, tail)], idx_t)
            pltpu.async_copy(pe_hbm.at[idx_t], rows_t, sem).wait()
            pltpu.sync_copy(rows_t, out_hbm.at[pl.ds(base, tail)])

    return k(pe, idx)


def _tc_add(seq, emb):
    b, s, d = seq.shape

    def body(a_ref, b_ref, o_ref):
        o_ref[...] = a_ref[...] + b_ref[...]

    return pl.pallas_call(
        body,
        grid=(b,),
        in_specs=[pl.BlockSpec((1, s, d), lambda i: (i, 0, 0))] * 2,
        out_specs=pl.BlockSpec((1, s, d), lambda i: (i, 0, 0)),
        out_shape=jax.ShapeDtypeStruct((b, s, d), jnp.float32),
    )(seq, emb)


def kernel(seq, coords, seq_id):
    b, s, d = seq.shape
    idx = jnp.concatenate(
        [coords.astype(jnp.int32), seq_id[..., None].astype(jnp.int32)], axis=-1
    )
    idx = jnp.clip(idx, 0, MAX_LEN - 1)
    idx = jnp.concatenate([jnp.zeros((b, 1, 4), jnp.int32), idx], axis=1)
    idx = idx.reshape(b * s * 4)

    pe = jnp.asarray(_PE)
    emb = _sc_gather(pe, idx, b * s * 4).reshape(b, s, d)
    out = _tc_add(seq, emb)
    return (out, emb)
